# baseline (speedup 1.0000x reference)
import functools

import jax
import jax.numpy as jnp
from jax import lax
from jax.experimental import pallas as pl
from jax.experimental.pallas import tpu as pltpu
from jax.experimental.pallas import tpu_sc as plsc

NW = 32
CHUNK = 128


def _sc_gather_fn(B, ED, CH, c):
    Bc = B // CH
    R = Bc // NW
    NC = R // CHUNK
    COFF = c * (Bc // CHUNK)
    f32 = jnp.float32
    mesh = plsc.VectorSubcoreMesh(core_axis_name="c", subcore_axis_name="s")

    @functools.partial(
        pl.kernel,
        out_type=(
            jax.ShapeDtypeStruct((Bc, ED), f32),
            jax.ShapeDtypeStruct((Bc, ED), f32),
            jax.ShapeDtypeStruct((Bc // CHUNK, CHUNK), f32),
        ),
        mesh=mesh,
        scratch_types=[
            pltpu.VMEM((NC, CHUNK), jnp.int32),
            pltpu.VMEM((NC, CHUNK), jnp.int32),
            pltpu.VMEM((R, 128), f32),
            pltpu.VMEM((NC, CHUNK), f32),
            pltpu.VMEM((NC, CHUNK), f32),
            pltpu.VMEM((NC, CHUNK), f32),
            pltpu.VMEM((16,), f32),
            pltpu.SemaphoreType.DMA,
        ],
        compiler_params=pltpu.CompilerParams(use_tc_tiling_on_sc=False),
    )
    def sc(uidx, iidx, uemb, iemb, wubp, wibp, gb,
           out_u, out_i, out_w,
           vu, vi, rows, bu, bi, wsum, gbv, sem):
        wid = lax.axis_index("s") * 2 + lax.axis_index("c")
        base = wid * R
        cbase = wid * NC
        gbase = COFF + cbase

        pltpu.sync_copy(uidx.at[pl.ds(gbase, NC)], vu)
        pltpu.sync_copy(iidx.at[pl.ds(gbase, NC)], vi)
        pltpu.sync_copy(gb, gbv)

        def gather_rows(tbl, idx_ref):
            cps = [pltpu.async_copy(tbl.at[idx_ref.at[j]],
                                    rows.at[pl.ds(j * CHUNK, CHUNK)], sem)
                   for j in range(NC)]
            for c in cps:
                c.wait()

        gather_rows(uemb, vu)
        pltpu.sync_copy(rows, out_u.at[pl.ds(base, R)])
        gather_rows(iemb, vi)
        pltpu.sync_copy(rows, out_i.at[pl.ds(base, R)])

        cps = [pltpu.async_copy(wubp.at[vu.at[j]], bu.at[j], sem)
               for j in range(NC)]
        cps += [pltpu.async_copy(wibp.at[vi.at[j]], bi.at[j], sem)
                for j in range(NC)]
        for c in cps:
            c.wait()

        gvec = gbv[...]
        for j in range(NC):
            for i in range(CHUNK // 16):
                s = pl.ds(i * 16, 16)
                wsum[j, s] = bu[j, s] + bi[j, s] + gvec
        pltpu.sync_copy(wsum, out_w.at[pl.ds(cbase, NC)])

    return sc


def _dotT(x, w):
    return lax.dot_general(x, w, (((1,), (1,)), ((), ())),
                           preferred_element_type=jnp.float32)


def _dotTb(x, w):
    return lax.dot_general(x.astype(jnp.bfloat16), w,
                           (((1,), (1,)), ((), ())),
                           preferred_element_type=jnp.float32)


def kernel(user_idx, item_idx, gender_idx, age_idx, occupation_idx,
           item_genre_features, global_bias, wide_user_bias, wide_item_bias,
           wide_gender_bias, wide_age_bias, wide_occupation_bias, wide_genre_W,
           user_emb, item_emb, gender_emb, age_emb, occupation_emb,
           genre_proj_W, genre_proj_b, W1, b1, W2, b2, W3, b3):
    B = user_idx.shape[0]
    ED = user_emb.shape[1]
    ED4 = gender_emb.shape[1]
    GED = genre_proj_W.shape[0]
    NG = item_genre_features.shape[1]
    NGen, NAge, NOcc = gender_emb.shape[0], age_emb.shape[0], occupation_emb.shape[0]
    H1, H2 = W1.shape[0], W2.shape[0]
    i32, f32 = jnp.int32, jnp.float32

    ui = user_idx.astype(i32).reshape(B // CHUNK, CHUNK)
    ii = item_idx.astype(i32).reshape(B // CHUNK, CHUNK)
    gb = jnp.broadcast_to((global_bias + b3).astype(f32), (16,))

    CH = 2
    Bc = B // CH
    sc_outs = [
        _sc_gather_fn(B, ED, CH, c)(
            ui, ii, user_emb, item_emb,
            wide_user_bias.reshape(-1), wide_item_bias.reshape(-1), gb)
        for c in range(CH)
    ]

    BM = 1024
    NBc = Bc // BM
    RB = BM // CHUNK
    bf16 = jnp.bfloat16

    gext = jnp.concatenate([gender_emb, wide_gender_bias], axis=1)
    aext = jnp.concatenate([age_emb, wide_age_bias], axis=1)
    oext = jnp.concatenate([occupation_emb, wide_occupation_bias], axis=1)
    gidx = gender_idx.astype(i32).reshape(B, 1)
    aidx = age_idx.astype(i32).reshape(B, 1)
    oidx = occupation_idx.astype(i32).reshape(B, 1)

    def tc_body(xu_r, xi_r, gi_r, ai_r, oi_r, genre_r, wide_r,
                gext_r, aext_r, oext_r, w1_r, b1_r,
                w2_r, b2_r, w3_r, gpw_r, gpb_r, wgw_r, out_r,
                w1b_r, w2b_r):
        @pl.when(pl.program_id(0) == 0)
        def _cast_weights():
            w1b_r[...] = w1_r[...].astype(bf16)
            w2b_r[...] = w2_r[...].astype(bf16)

        g = genre_r[...]

        def emb_lookup(idx_r, ext_r, ncls):
            oh = (idx_r[...] == lax.broadcasted_iota(i32, (BM, ncls), 1))
            x = jnp.dot(oh.astype(f32), ext_r[...],
                        preferred_element_type=f32)
            return x[:, :ED4], x[:, ED4:]

        xg, wbg = emb_lookup(gi_r, gext_r, NGen)
        xa, wba = emb_lookup(ai_r, aext_r, NAge)
        xo, wbo = emb_lookup(oi_r, oext_r, NOcc)

        x_gen = _dotT(g, gpw_r[...]) + gpb_r[...]
        xs = jnp.concatenate([xg, xa, xo, x_gen], axis=1)
        h1 = (_dotTb(xu_r[...], w1b_r[:, :ED]) +
              _dotTb(xi_r[...], w1b_r[:, ED:2 * ED]) +
              _dotTb(xs, w1b_r[:, 2 * ED:]) + b1_r[...])
        h1 = jnp.maximum(h1, 0.0)
        h2 = jnp.maximum(_dotTb(h1, w2b_r[...]) + b2_r[...], 0.0)
        deep = _dotT(h2, w3_r[...])
        wide_g = _dotT(g, wgw_r[...])
        out_r[...] = wide_r[...] + wide_g + wbg + wba + wbo + deep

    def rows(minor):
        return pl.BlockSpec((BM, minor), lambda i: (i, 0))

    def whole(a):
        return pl.BlockSpec(a.shape, lambda i: (0,) * a.ndim)

    b1r, b2r = b1.reshape(1, H1), b2.reshape(1, H2)
    gpbr = genre_proj_b.reshape(1, GED)

    outs = []
    for c in range(CH):
        xu, xi, widev = sc_outs[c]
        wide2 = widev.reshape(Bc, 1)
        operands = (xu, xi, gidx, aidx, oidx, item_genre_features, wide2,
                    gext, aext, oext, W1, b1r,
                    W2, b2r, W3, genre_proj_W, gpbr, wide_genre_W)

        def off(minor, c=c):
            return pl.BlockSpec((BM, minor), lambda i: (c * NBc + i, 0))

        in_specs = [rows(ED), rows(ED), off(1), off(1), off(1), off(NG),
                    rows(1)]
        in_specs += [whole(a) for a in operands[7:]]

        outs.append(pl.pallas_call(
            tc_body,
            grid=(NBc,),
            in_specs=in_specs,
            out_specs=pl.BlockSpec((BM, 1), lambda i: (i, 0)),
            out_shape=jax.ShapeDtypeStruct((Bc, 1), f32),
            scratch_shapes=[pltpu.VMEM((H1, W1.shape[1]), bf16),
                            pltpu.VMEM((H2, H1), bf16)],
        )(*operands))
    return jnp.concatenate([o.reshape(Bc) for o in outs], axis=0)


# speedup vs baseline: 1.0844x; 1.0844x over previous
import functools

import jax
import jax.numpy as jnp
from jax import lax
from jax.experimental import pallas as pl
from jax.experimental.pallas import tpu as pltpu
from jax.experimental.pallas import tpu_sc as plsc

NW = 32
CHUNK = 128


def _sc_gather_fn(B, ED):
    R = B // NW
    NC = R // CHUNK
    f32 = jnp.float32
    mesh = plsc.VectorSubcoreMesh(core_axis_name="c", subcore_axis_name="s")

    @functools.partial(
        pl.kernel,
        out_type=(
            jax.ShapeDtypeStruct((B, ED), f32),
            jax.ShapeDtypeStruct((B, ED), f32),
            jax.ShapeDtypeStruct((B // CHUNK, CHUNK), f32),
        ),
        mesh=mesh,
        scratch_types=[
            pltpu.VMEM((NC, CHUNK), jnp.int32),
            pltpu.VMEM((NC, CHUNK), jnp.int32),
            pltpu.VMEM((R, 128), f32),
            pltpu.VMEM((NC, CHUNK), f32),
            pltpu.VMEM((NC, CHUNK), f32),
            pltpu.VMEM((NC, CHUNK), f32),
            pltpu.VMEM((16,), f32),
            pltpu.SemaphoreType.DMA,
        ],
        compiler_params=pltpu.CompilerParams(use_tc_tiling_on_sc=False),
    )
    def sc(uidx, iidx, uemb, iemb, wubp, wibp, gb,
           out_u, out_i, out_w,
           vu, vi, rows, bu, bi, wsum, gbv, sem):
        wid = lax.axis_index("s") * 2 + lax.axis_index("c")
        base = wid * R
        cbase = wid * NC

        pltpu.sync_copy(uidx.at[pl.ds(cbase, NC)], vu)
        pltpu.sync_copy(iidx.at[pl.ds(cbase, NC)], vi)
        pltpu.sync_copy(gb, gbv)

        def gather_rows(tbl, idx_ref):
            cps = [pltpu.async_copy(tbl.at[idx_ref.at[j]],
                                    rows.at[pl.ds(j * CHUNK, CHUNK)], sem)
                   for j in range(NC)]
            for c in cps:
                c.wait()

        gather_rows(uemb, vu)
        pltpu.sync_copy(rows, out_u.at[pl.ds(base, R)])
        gather_rows(iemb, vi)
        pltpu.sync_copy(rows, out_i.at[pl.ds(base, R)])

        cps = [pltpu.async_copy(wubp.at[vu.at[j]], bu.at[j], sem)
               for j in range(NC)]
        cps += [pltpu.async_copy(wibp.at[vi.at[j]], bi.at[j], sem)
                for j in range(NC)]
        for c in cps:
            c.wait()

        gvec = gbv[...]
        for j in range(NC):
            for i in range(CHUNK // 16):
                s = pl.ds(i * 16, 16)
                wsum[j, s] = bu[j, s] + bi[j, s] + gvec
        pltpu.sync_copy(wsum, out_w.at[pl.ds(cbase, NC)])

    return sc


def _dotT(x, w):
    return lax.dot_general(x, w, (((1,), (1,)), ((), ())),
                           preferred_element_type=jnp.float32)


def _dotTb(x, w, out_dtype=jnp.float32):
    return lax.dot_general(x.astype(jnp.bfloat16), w,
                           (((1,), (1,)), ((), ())),
                           preferred_element_type=out_dtype)


def kernel(user_idx, item_idx, gender_idx, age_idx, occupation_idx,
           item_genre_features, global_bias, wide_user_bias, wide_item_bias,
           wide_gender_bias, wide_age_bias, wide_occupation_bias, wide_genre_W,
           user_emb, item_emb, gender_emb, age_emb, occupation_emb,
           genre_proj_W, genre_proj_b, W1, b1, W2, b2, W3, b3):
    B = user_idx.shape[0]
    ED = user_emb.shape[1]
    ED4 = gender_emb.shape[1]
    GED = genre_proj_W.shape[0]
    NG = item_genre_features.shape[1]
    NGen, NAge, NOcc = gender_emb.shape[0], age_emb.shape[0], occupation_emb.shape[0]
    H1, H2 = W1.shape[0], W2.shape[0]
    i32, f32 = jnp.int32, jnp.float32

    ui = user_idx.astype(i32).reshape(B // CHUNK, CHUNK)
    ii = item_idx.astype(i32).reshape(B // CHUNK, CHUNK)
    gb = jnp.broadcast_to((global_bias + b3).astype(f32), (16,))

    xu, xi, widev = _sc_gather_fn(B, ED)(
        ui, ii, user_emb, item_emb,
        wide_user_bias.reshape(-1), wide_item_bias.reshape(-1), gb)

    BM = 2048
    NB = B // BM
    bf16 = jnp.bfloat16

    gext = jnp.concatenate([gender_emb, wide_gender_bias], axis=1)
    aext = jnp.concatenate([age_emb, wide_age_bias], axis=1)
    oext = jnp.concatenate([occupation_emb, wide_occupation_bias], axis=1)
    gidx = gender_idx.astype(i32).reshape(B, 1)
    aidx = age_idx.astype(i32).reshape(B, 1)
    oidx = occupation_idx.astype(i32).reshape(B, 1)

    def tc_body(xu_r, xi_r, gi_r, ai_r, oi_r, genre_r, wide_r,
                gext_r, aext_r, oext_r, w1_r, b1_r,
                w2_r, b2_r, w3_r, gpw_r, gpb_r, wgw_r, out_r,
                w1b_r, w2b_r):
        @pl.when(pl.program_id(0) == 0)
        def _cast_weights():
            w1b_r[...] = w1_r[...].astype(bf16)
            w2b_r[...] = w2_r[...].astype(bf16)

        g = genre_r[...]

        def emb_lookup(idx_r, ext_r, ncls):
            oh = (idx_r[...] == lax.broadcasted_iota(i32, (BM, ncls), 1))
            x = jnp.dot(oh.astype(f32), ext_r[...],
                        preferred_element_type=f32)
            return x[:, :ED4], x[:, ED4:]

        xg, wbg = emb_lookup(gi_r, gext_r, NGen)
        xa, wba = emb_lookup(ai_r, aext_r, NAge)
        xo, wbo = emb_lookup(oi_r, oext_r, NOcc)

        x_gen = _dotT(g, gpw_r[...]) + gpb_r[...]
        xs = jnp.concatenate([xg, xa, xo, x_gen], axis=1)
        h1 = (_dotTb(xu_r[...], w1b_r[:, :ED]) +
              _dotTb(xi_r[...], w1b_r[:, ED:2 * ED]) +
              _dotTb(xs, w1b_r[:, 2 * ED:]) + b1_r[...])
        h1 = jnp.maximum(h1, 0.0)
        h2 = jnp.maximum(_dotTb(h1, w2b_r[...]) + b2_r[...], 0.0)
        deep = _dotT(h2, w3_r[...])
        wide_g = _dotT(g, wgw_r[...])
        out_r[...] = wide_r[...] + wide_g + wbg + wba + wbo + deep

    def rows(minor):
        return pl.BlockSpec((BM, minor), lambda i: (i, 0))

    def whole(a):
        return pl.BlockSpec(a.shape, lambda i: (0,) * a.ndim)

    b1r, b2r = b1.reshape(1, H1), b2.reshape(1, H2)
    gpbr = genre_proj_b.reshape(1, GED)
    wide2 = widev.reshape(B, 1)
    operands = (xu, xi, gidx, aidx, oidx, item_genre_features, wide2,
                gext, aext, oext, W1, b1r,
                W2, b2r, W3, genre_proj_W, gpbr, wide_genre_W)
    in_specs = [rows(ED), rows(ED), rows(1), rows(1), rows(1), rows(NG),
                rows(1)]
    in_specs += [whole(a) for a in operands[7:]]

    out = pl.pallas_call(
        tc_body,
        grid=(NB,),
        in_specs=in_specs,
        out_specs=pl.BlockSpec((BM, 1), lambda i: (i, 0)),
        out_shape=jax.ShapeDtypeStruct((B, 1), f32),
        scratch_shapes=[pltpu.VMEM((H1, W1.shape[1]), bf16),
                        pltpu.VMEM((H2, H1), bf16)],
    )(*operands)
    return out.reshape(B)


# speedup vs baseline: 1.0963x; 1.0109x over previous
import functools

import jax
import jax.numpy as jnp
from jax import lax
from jax.experimental import pallas as pl
from jax.experimental.pallas import tpu as pltpu
from jax.experimental.pallas import tpu_sc as plsc

NW = 32
CHUNK = 128


def _sc_gather_fn(B, ED):
    R = B // NW
    NC = R // CHUNK
    f32 = jnp.float32
    mesh = plsc.VectorSubcoreMesh(core_axis_name="c", subcore_axis_name="s")

    @functools.partial(
        pl.kernel,
        out_type=(
            jax.ShapeDtypeStruct((B, ED), f32),
            jax.ShapeDtypeStruct((B, ED), f32),
            jax.ShapeDtypeStruct((B // CHUNK, CHUNK), f32),
        ),
        mesh=mesh,
        scratch_types=[
            pltpu.VMEM((NC, CHUNK), jnp.int32),
            pltpu.VMEM((NC, CHUNK), jnp.int32),
            pltpu.VMEM((R, 128), f32),
            pltpu.VMEM((NC, CHUNK), f32),
            pltpu.VMEM((NC, CHUNK), f32),
            pltpu.VMEM((NC, CHUNK), f32),
            pltpu.VMEM((16,), f32),
            pltpu.SemaphoreType.DMA,
        ],
        compiler_params=pltpu.CompilerParams(use_tc_tiling_on_sc=False),
    )
    def sc(uidx, iidx, uemb, iemb, wubp, wibp, gb,
           out_u, out_i, out_w,
           vu, vi, rows, bu, bi, wsum, gbv, sem):
        wid = lax.axis_index("s") * 2 + lax.axis_index("c")
        base = wid * R
        cbase = wid * NC

        pltpu.sync_copy(uidx.at[pl.ds(cbase, NC)], vu)
        pltpu.sync_copy(iidx.at[pl.ds(cbase, NC)], vi)
        pltpu.sync_copy(gb, gbv)

        def gather_rows(tbl, idx_ref):
            cps = [pltpu.async_copy(tbl.at[idx_ref.at[j]],
                                    rows.at[pl.ds(j * CHUNK, CHUNK)], sem)
                   for j in range(NC)]
            for c in cps:
                c.wait()

        gather_rows(uemb, vu)
        pltpu.sync_copy(rows, out_u.at[pl.ds(base, R)])
        gather_rows(iemb, vi)
        pltpu.sync_copy(rows, out_i.at[pl.ds(base, R)])

        cps = [pltpu.async_copy(wubp.at[vu.at[j]], bu.at[j], sem)
               for j in range(NC)]
        cps += [pltpu.async_copy(wibp.at[vi.at[j]], bi.at[j], sem)
                for j in range(NC)]
        for c in cps:
            c.wait()

        gvec = gbv[...]
        for j in range(NC):
            for i in range(CHUNK // 16):
                s = pl.ds(i * 16, 16)
                wsum[j, s] = bu[j, s] + bi[j, s] + gvec
        pltpu.sync_copy(wsum, out_w.at[pl.ds(cbase, NC)])

    return sc


def _dotT(x, w):
    return lax.dot_general(x, w, (((1,), (1,)), ((), ())),
                           preferred_element_type=jnp.float32)


def _dotTb(x, w, out_dtype=jnp.float32):
    return lax.dot_general(x.astype(jnp.bfloat16), w,
                           (((1,), (1,)), ((), ())),
                           preferred_element_type=out_dtype)


def kernel(user_idx, item_idx, gender_idx, age_idx, occupation_idx,
           item_genre_features, global_bias, wide_user_bias, wide_item_bias,
           wide_gender_bias, wide_age_bias, wide_occupation_bias, wide_genre_W,
           user_emb, item_emb, gender_emb, age_emb, occupation_emb,
           genre_proj_W, genre_proj_b, W1, b1, W2, b2, W3, b3):
    B = user_idx.shape[0]
    ED = user_emb.shape[1]
    ED4 = gender_emb.shape[1]
    GED = genre_proj_W.shape[0]
    NG = item_genre_features.shape[1]
    NGen, NAge, NOcc = gender_emb.shape[0], age_emb.shape[0], occupation_emb.shape[0]
    H1, H2 = W1.shape[0], W2.shape[0]
    i32, f32 = jnp.int32, jnp.float32

    ui = user_idx.astype(i32).reshape(B // CHUNK, CHUNK)
    ii = item_idx.astype(i32).reshape(B // CHUNK, CHUNK)
    gb = jnp.broadcast_to((global_bias + b3).astype(f32), (16,))

    xu, xi, widev = _sc_gather_fn(B, ED)(
        ui, ii, user_emb, item_emb,
        wide_user_bias.reshape(-1), wide_item_bias.reshape(-1), gb)

    BM = 2048
    NB = B // BM
    bf16 = jnp.bfloat16

    gext = jnp.concatenate([gender_emb, wide_gender_bias], axis=1)
    aext = jnp.concatenate([age_emb, wide_age_bias], axis=1)
    oext = jnp.concatenate([occupation_emb, wide_occupation_bias], axis=1)
    gidx = gender_idx.astype(i32).reshape(B, 1)
    aidx = age_idx.astype(i32).reshape(B, 1)
    oidx = occupation_idx.astype(i32).reshape(B, 1)

    def tc_body(xu_r, xi_r, gi_r, ai_r, oi_r, genre_r, wide_r,
                gext_r, aext_r, oext_r, w1b_r, b1_r,
                w2b_r, b2_r, w3_r, gpw_r, gpb_r, wgw_r, out_r):
        g = genre_r[...].astype(f32)

        def emb_lookup(idx_r, ext_r, ncls):
            oh = (idx_r[...] == lax.broadcasted_iota(i32, (BM, ncls), 1))
            x = jnp.dot(oh.astype(f32), ext_r[...],
                        preferred_element_type=f32)
            return x[:, :ED4], x[:, ED4:]

        xg, wbg = emb_lookup(gi_r, gext_r, NGen)
        xa, wba = emb_lookup(ai_r, aext_r, NAge)
        xo, wbo = emb_lookup(oi_r, oext_r, NOcc)

        x_gen = _dotT(g, gpw_r[...]) + gpb_r[...]
        xs = jnp.concatenate([xg, xa, xo, x_gen], axis=1)
        h1 = (_dotTb(xu_r[...], w1b_r[:, :ED]) +
              _dotTb(xi_r[...], w1b_r[:, ED:2 * ED]) +
              _dotTb(xs, w1b_r[:, 2 * ED:]) + b1_r[...])
        h1 = jnp.maximum(h1, 0.0)
        h2 = jnp.maximum(_dotTb(h1, w2b_r[...]) + b2_r[...], 0.0)
        deep = _dotT(h2, w3_r[...])
        wide_g = _dotT(g, wgw_r[...])
        out_r[...] = wide_r[...] + wide_g + wbg + wba + wbo + deep

    def rows(minor):
        return pl.BlockSpec((BM, minor), lambda i: (i, 0))

    def whole(a):
        return pl.BlockSpec(a.shape, lambda i: (0,) * a.ndim)

    b1r, b2r = b1.reshape(1, H1), b2.reshape(1, H2)
    gpbr = genre_proj_b.reshape(1, GED)
    wide2 = widev.reshape(B, 1)
    W1b, W2b = W1.astype(bf16), W2.astype(bf16)
    genre_b = item_genre_features.astype(bf16)
    operands = (xu, xi, gidx, aidx, oidx, genre_b, wide2,
                gext, aext, oext, W1b, b1r,
                W2b, b2r, W3, genre_proj_W, gpbr, wide_genre_W)
    in_specs = [rows(ED), rows(ED), rows(1), rows(1), rows(1), rows(NG),
                rows(1)]
    in_specs += [whole(a) for a in operands[7:]]

    out = pl.pallas_call(
        tc_body,
        grid=(NB,),
        in_specs=in_specs,
        out_specs=pl.BlockSpec((BM, 1), lambda i: (i, 0)),
        out_shape=jax.ShapeDtypeStruct((B, 1), f32),
    )(*operands)
    return out.reshape(B)


# speedup vs baseline: 1.1452x; 1.0447x over previous
import functools

import jax
import jax.numpy as jnp
from jax import lax
from jax.experimental import pallas as pl
from jax.experimental.pallas import tpu as pltpu
from jax.experimental.pallas import tpu_sc as plsc

NW = 32
CHUNK = 128


def _sc_gather_fn(B, ED):
    R = B // NW
    NC = R // CHUNK
    f32 = jnp.float32
    mesh = plsc.VectorSubcoreMesh(core_axis_name="c", subcore_axis_name="s")

    @functools.partial(
        pl.kernel,
        out_type=(
            jax.ShapeDtypeStruct((B, ED), f32),
            jax.ShapeDtypeStruct((B, ED), f32),
            jax.ShapeDtypeStruct((B // CHUNK, CHUNK), f32),
        ),
        mesh=mesh,
        scratch_types=[
            pltpu.VMEM((NC, CHUNK), jnp.int32),
            pltpu.VMEM((NC, CHUNK), jnp.int32),
            pltpu.VMEM((R, 128), f32),
            pltpu.VMEM((NC, CHUNK), f32),
            pltpu.VMEM((NC, CHUNK), f32),
            pltpu.VMEM((NC, CHUNK), f32),
            pltpu.VMEM((16,), f32),
            pltpu.SemaphoreType.DMA,
        ],
        compiler_params=pltpu.CompilerParams(use_tc_tiling_on_sc=False),
    )
    def sc(uidx, iidx, uemb, iemb, wubp, wibp, gb,
           out_u, out_i, out_w,
           vu, vi, rows, bu, bi, wsum, gbv, sem):
        wid = lax.axis_index("s") * 2 + lax.axis_index("c")
        base = wid * R
        cbase = wid * NC

        pltpu.sync_copy(uidx.at[pl.ds(cbase, NC)], vu)
        pltpu.sync_copy(iidx.at[pl.ds(cbase, NC)], vi)
        pltpu.sync_copy(gb, gbv)

        def gather_rows(tbl, idx_ref):
            cps = [pltpu.async_copy(tbl.at[idx_ref.at[j]],
                                    rows.at[pl.ds(j * CHUNK, CHUNK)], sem)
                   for j in range(NC)]
            for c in cps:
                c.wait()

        gather_rows(uemb, vu)
        pltpu.sync_copy(rows, out_u.at[pl.ds(base, R)])
        gather_rows(iemb, vi)
        pltpu.sync_copy(rows, out_i.at[pl.ds(base, R)])

        cps = [pltpu.async_copy(wubp.at[vu.at[j]], bu.at[j], sem)
               for j in range(NC)]
        cps += [pltpu.async_copy(wibp.at[vi.at[j]], bi.at[j], sem)
                for j in range(NC)]
        for c in cps:
            c.wait()

        gvec = gbv[...]
        for j in range(NC):
            for i in range(CHUNK // 16):
                s = pl.ds(i * 16, 16)
                wsum[j, s] = bu[j, s] + bi[j, s] + gvec
        pltpu.sync_copy(wsum, out_w.at[pl.ds(cbase, NC)])

    return sc


def _dotT(x, w):
    return lax.dot_general(x, w, (((1,), (1,)), ((), ())),
                           preferred_element_type=jnp.float32)


def _dotTb(x, w, out_dtype=jnp.float32):
    return lax.dot_general(x.astype(jnp.bfloat16), w,
                           (((1,), (1,)), ((), ())),
                           preferred_element_type=out_dtype)


def kernel(user_idx, item_idx, gender_idx, age_idx, occupation_idx,
           item_genre_features, global_bias, wide_user_bias, wide_item_bias,
           wide_gender_bias, wide_age_bias, wide_occupation_bias, wide_genre_W,
           user_emb, item_emb, gender_emb, age_emb, occupation_emb,
           genre_proj_W, genre_proj_b, W1, b1, W2, b2, W3, b3):
    B = user_idx.shape[0]
    ED = user_emb.shape[1]
    ED4 = gender_emb.shape[1]
    GED = genre_proj_W.shape[0]
    NG = item_genre_features.shape[1]
    NGen, NAge, NOcc = gender_emb.shape[0], age_emb.shape[0], occupation_emb.shape[0]
    H1, H2 = W1.shape[0], W2.shape[0]
    i32, f32 = jnp.int32, jnp.float32

    ui = user_idx.astype(i32).reshape(B // CHUNK, CHUNK)
    ii = item_idx.astype(i32).reshape(B // CHUNK, CHUNK)
    gb = jnp.broadcast_to((global_bias + b3).astype(f32), (16,))

    xu, xi, widev = _sc_gather_fn(B, ED)(
        ui, ii, user_emb, item_emb,
        wide_user_bias.reshape(-1), wide_item_bias.reshape(-1), gb)

    BM = 2048
    NB = B // BM
    bf16 = jnp.bfloat16

    gext = jnp.concatenate([gender_emb, wide_gender_bias], axis=1)
    aext = jnp.concatenate([age_emb, wide_age_bias], axis=1)
    oext = jnp.concatenate([occupation_emb, wide_occupation_bias], axis=1)
    gidx = gender_idx.astype(jnp.int8).reshape(B, 1)
    aidx = age_idx.astype(jnp.int8).reshape(B, 1)
    oidx = occupation_idx.astype(jnp.int8).reshape(B, 1)

    def tc_body(xu_r, xi_r, gi_r, ai_r, oi_r, genre_r, wide_r,
                gext_r, aext_r, oext_r, w1b_r, b1_r,
                w2b_r, b2_r, w3_r, gpw_r, gpb_r, wgw_r, out_r):
        g = genre_r[...].astype(f32)

        def emb_lookup(idx_r, ext_r, ncls):
            oh = (idx_r[...].astype(i32)
                  == lax.broadcasted_iota(i32, (BM, ncls), 1))
            x = jnp.dot(oh.astype(f32), ext_r[...],
                        preferred_element_type=f32)
            return x[:, :ED4], x[:, ED4:]

        xg, wbg = emb_lookup(gi_r, gext_r, NGen)
        xa, wba = emb_lookup(ai_r, aext_r, NAge)
        xo, wbo = emb_lookup(oi_r, oext_r, NOcc)

        x_gen = _dotT(g, gpw_r[...]) + gpb_r[...]
        xs = jnp.concatenate([xg, xa, xo, x_gen], axis=1)
        h1 = (_dotTb(xu_r[...], w1b_r[:, :ED]) +
              _dotTb(xi_r[...], w1b_r[:, ED:2 * ED]) +
              _dotTb(xs, w1b_r[:, 2 * ED:]) + b1_r[...])
        h1 = jnp.maximum(h1, 0.0)
        h2 = jnp.maximum(_dotTb(h1, w2b_r[...]) + b2_r[...], 0.0)
        deep = _dotT(h2, w3_r[...])
        wide_g = _dotT(g, wgw_r[...])
        out_r[...] = wide_r[...].astype(f32) + wide_g + wbg + wba + wbo + deep

    def rows(minor):
        return pl.BlockSpec((BM, minor), lambda i: (i, 0))

    def whole(a):
        return pl.BlockSpec(a.shape, lambda i: (0,) * a.ndim)

    b1r, b2r = b1.reshape(1, H1), b2.reshape(1, H2)
    gpbr = genre_proj_b.reshape(1, GED)
    wide2 = widev.astype(bf16).reshape(B, 1)
    W1b, W2b = W1.astype(bf16), W2.astype(bf16)
    genre_b = item_genre_features.astype(bf16)
    operands = (xu, xi, gidx, aidx, oidx, genre_b, wide2,
                gext, aext, oext, W1b, b1r,
                W2b, b2r, W3, genre_proj_W, gpbr, wide_genre_W)
    in_specs = [rows(ED), rows(ED), rows(1), rows(1), rows(1), rows(NG),
                rows(1)]
    in_specs += [whole(a) for a in operands[7:]]

    out = pl.pallas_call(
        tc_body,
        grid=(NB,),
        in_specs=in_specs,
        out_specs=pl.BlockSpec((BM, 1), lambda i: (i, 0)),
        out_shape=jax.ShapeDtypeStruct((B, 1), f32),
    )(*operands)
    return out.reshape(B)
